# baseline (speedup 1.0000x reference)
import functools

import jax
import jax.numpy as jnp
from jax import lax
from jax.experimental import pallas as pl
from jax.experimental.pallas import tpu as pltpu
from jax.experimental.pallas import tpu_sc as plsc

N = 10000
E = 160000
D = 256
H = 128
NSC = 2
NT = 16
CH = 128
CHUNKS = 80
NBUF = 3
E_PAD = NT * CHUNKS * CH
N_PAD = 10112
ZROWS = N_PAD // NT

_SC_MESH = plsc.VectorSubcoreMesh(core_axis_name="c", subcore_axis_name="s")


@functools.partial(
    pl.kernel,
    out_type=jax.ShapeDtypeStruct((NSC, N_PAD, H), jnp.float32),
    mesh=_SC_MESH,
    scratch_types=[
        pltpu.VMEM((NBUF, 2, CH), jnp.int32),
        pltpu.VMEM((NBUF, CH, H), jnp.float32),
        pltpu.VMEM_SHARED((N_PAD, H), jnp.float32),
        pltpu.SemaphoreType.DMA((NBUF,)),
        pltpu.SemaphoreType.DMA((NBUF,)),
        pltpu.SemaphoreType.DMA((NBUF,)),
    ],
)
def _sc_agg(tbl, sd3, zeros, out, sd, rows, acc, isems, sems, ssems):
    c = lax.axis_index("c")
    s = lax.axis_index("s")
    pltpu.sync_copy(zeros, acc.at[pl.ds(s * ZROWS, ZROWS)])
    plsc.subcore_barrier()

    cbase = s * CHUNKS

    def _issue_idx(i, slot):
        pltpu.async_copy(sd3.at[c, cbase + i], sd.at[slot], isems.at[slot])

    def _issue_gather(i, slot):
        pltpu.make_async_copy(sd3.at[c, cbase + i], sd.at[slot],
                              isems.at[slot]).wait()
        pltpu.async_copy(tbl.at[sd.at[slot, 0]], rows.at[slot],
                         sems.at[slot])

    def _wait_scatter(slot):
        pltpu.make_async_copy(rows.at[slot], acc.at[sd.at[slot, 1]],
                              ssems.at[slot]).wait()

    def _body(i, slot, nslot, first):
        if first:
            @pl.when(i >= 2)
            def _():
                _wait_scatter(nslot)
        else:
            _wait_scatter(nslot)
        _issue_idx(i + 1, nslot)
        pltpu.make_async_copy(tbl.at[sd.at[slot, 0]], rows.at[slot],
                              sems.at[slot]).wait()
        pltpu.async_copy(rows.at[slot], acc.at[sd.at[slot, 1]],
                         ssems.at[slot], add=True)
        _issue_gather(i + 1, nslot)

    pltpu.sync_copy(sd3.at[c, cbase], sd.at[0])
    pltpu.async_copy(tbl.at[sd.at[0, 0]], rows.at[0], sems.at[0])

    @pl.loop(0, CHUNKS - 2, step=NBUF)
    def _(g):
        for off in range(NBUF):
            _body(g + off, off, (off + 1) % NBUF, first=True)

    i0 = CHUNKS - 2
    _body(i0, i0 % NBUF, (i0 + 1) % NBUF, first=False)
    i1 = CHUNKS - 1
    _wait_scatter((i1 + 1) % NBUF)
    pltpu.make_async_copy(tbl.at[sd.at[i1 % NBUF, 0]], rows.at[i1 % NBUF],
                          sems.at[i1 % NBUF]).wait()
    pltpu.async_copy(rows.at[i1 % NBUF], acc.at[sd.at[i1 % NBUF, 1]],
                     ssems.at[i1 % NBUF], add=True)
    _wait_scatter(i0 % NBUF)
    _wait_scatter(i1 % NBUF)

    plsc.subcore_barrier()
    pltpu.sync_copy(acc.at[pl.ds(s * ZROWS, ZROWS)],
                    out.at[c, pl.ds(s * ZROWS, ZROWS)])


_R = 400


def _tc_body(relu_after, xs_ref, agg_ref, wa_ref, ba_ref, wb_ref, bb_ref,
             out_ref):
    h = jnp.concatenate(
        [xs_ref[0] + agg_ref[0], xs_ref[1] + agg_ref[1]], axis=-1)
    h = jnp.dot(h, wa_ref[...], preferred_element_type=jnp.float32)
    h = jnp.maximum(h + ba_ref[...], 0.0)
    h = jnp.dot(h, wb_ref[...], preferred_element_type=jnp.float32)
    h = h + bb_ref[...]
    if relu_after:
        h = jnp.maximum(h, 0.0)
        out_ref[0] = h[:, :H]
        out_ref[1] = h[:, H:]
    else:
        out_ref[...] = h


def _tc_mlp_call(relu_after, xs, agg, wa, ba, wb, bb):
    grid = N // _R
    split_spec = pl.BlockSpec((NSC, _R, H), lambda i: (0, i, 0))
    full = pl.BlockSpec((D, D), lambda i: (0, 0))
    bias = pl.BlockSpec((1, D), lambda i: (0, 0))
    if relu_after:
        out_shape = jax.ShapeDtypeStruct((NSC, N, H), jnp.float32)
        out_spec = split_spec
    else:
        out_shape = jax.ShapeDtypeStruct((N, D), jnp.float32)
        out_spec = pl.BlockSpec((_R, D), lambda i: (i, 0))
    return pl.pallas_call(
        functools.partial(_tc_body, relu_after),
        grid=(grid,),
        in_specs=[split_spec, split_spec, full, bias, full, bias],
        out_specs=out_spec,
        out_shape=out_shape,
    )(xs, agg, wa, ba, wb, bb)


def kernel(x, edge_index, W1a, b1a, W1b, b1b, W2a, b2a, W2b, b2b):
    src = edge_index[0]
    dst = edge_index[1]
    pad = E_PAD - E
    srcp = jnp.concatenate([src, jnp.zeros((pad,), jnp.int32)])
    dstp = jnp.concatenate([dst, jnp.full((pad,), N, jnp.int32)])
    dstc = dstp.reshape(NT * CHUNKS, CH)
    sd3 = jnp.stack([
        jnp.stack([(srcp + c * N).reshape(NT * CHUNKS, CH), dstc], axis=1)
        for c in range(NSC)])
    zeros = jnp.zeros((ZROWS, H), jnp.float32)

    x2 = jnp.stack([x[:, :H], x[:, H:]])
    agg1 = _sc_agg(x2.reshape(NSC * N, H), sd3, zeros)
    h2 = _tc_mlp_call(True, x2, agg1, W1a, b1a.reshape(1, D), W1b,
                      b1b.reshape(1, D))
    agg2 = _sc_agg(h2.reshape(NSC * N, H), sd3, zeros)
    out = _tc_mlp_call(False, h2, agg2, W2a, b2a.reshape(1, D), W2b,
                       b2b.reshape(1, D))
    return out
